# Initial kernel scaffold; baseline (speedup 1.0000x reference)
import functools

import jax
import jax.numpy as jnp
from jax import lax
from jax.experimental import pallas as pl
from jax.experimental.pallas import tpu as pltpu
from jax.experimental.pallas import tpu_sc as plsc

N = 50000
D = 256
S = 512
BLK = 80
NBLK = N // BLK
NC, NS = 2, 16
NW = NC * NS
ITERS = -(-NBLK // NW)
CW = 16
RPT = S // NS


def _sc_partials(x, batch_i32):
    mesh = plsc.VectorSubcoreMesh(core_axis_name="c", subcore_axis_name="s")

    @functools.partial(
        pl.kernel,
        mesh=mesh,
        out_type=(
            jax.ShapeDtypeStruct((NC, S, D), jnp.float32),
            jax.ShapeDtypeStruct((NC, S, CW), jnp.float32),
        ),
        scratch_types=[
            pltpu.VMEM((BLK,), jnp.int32),
            pltpu.VMEM((BLK, D), jnp.float32),
            pltpu.VMEM((BLK, CW), jnp.float32),
            pltpu.VMEM((RPT, D), jnp.float32),
            pltpu.VMEM((RPT, CW), jnp.float32),
            pltpu.VMEM_SHARED((S, D), jnp.float32),
            pltpu.VMEM_SHARED((S, CW), jnp.float32),
        ],
    )
    def k(x_hbm, b_hbm, sums_out, cnts_out, idx_v, x_v, ones_v, z_v, zc_v,
          sums_sh, cnts_sh):
        c = lax.axis_index("c")
        s = lax.axis_index("s")
        wid = c * NS + s

        def fill_rows(i, _):
            def fill_lanes(j, _):
                z_v[i, pl.ds(j * 16, 16)] = jnp.zeros((16,), jnp.float32)
                return 0
            lax.fori_loop(0, D // 16, fill_lanes, 0)
            zc_v[i] = jnp.zeros((CW,), jnp.float32)
            return 0
        lax.fori_loop(0, RPT, fill_rows, 0)

        def fill_ones(i, _):
            ones_v[i] = jnp.ones((CW,), jnp.float32)
            return 0
        lax.fori_loop(0, BLK, fill_ones, 0)

        pltpu.sync_copy(z_v, sums_sh.at[pl.ds(s * RPT, RPT)])
        pltpu.sync_copy(zc_v, cnts_sh.at[pl.ds(s * RPT, RPT)])
        plsc.subcore_barrier()

        for i in range(ITERS):
            b = i * NW + wid

            @pl.when(b < NBLK)
            def _():
                base = b * BLK
                pltpu.sync_copy(b_hbm.at[pl.ds(base, BLK)], idx_v)
                pltpu.sync_copy(x_hbm.at[pl.ds(base, BLK)], x_v)
                pltpu.sync_copy(x_v, sums_sh.at[idx_v], add=True)
                pltpu.sync_copy(ones_v, cnts_sh.at[idx_v], add=True)

        plsc.subcore_barrier()

        pltpu.sync_copy(sums_sh.at[pl.ds(s * RPT, RPT)], z_v)
        pltpu.sync_copy(z_v, sums_out.at[c, pl.ds(s * RPT, RPT)])
        pltpu.sync_copy(cnts_sh.at[pl.ds(s * RPT, RPT)], zc_v)
        pltpu.sync_copy(zc_v, cnts_out.at[c, pl.ds(s * RPT, RPT)])

    return k(x, batch_i32)


def _combine_body(p_ref, c_ref, o_ref):
    sums = p_ref[0] + p_ref[1]
    cnts = c_ref[0] + c_ref[1]
    cnt = jnp.maximum(cnts[:, 0:1], 1.0)
    o_ref[...] = sums / cnt


def kernel(x, batch):
    batch_i32 = batch.astype(jnp.int32)
    partial_sums, partial_cnts = _sc_partials(x, batch_i32)
    return pl.pallas_call(
        _combine_body,
        out_shape=jax.ShapeDtypeStruct((S, D), jnp.float32),
    )(partial_sums, partial_cnts)


# speedup vs baseline: 7.6528x; 7.6528x over previous
import functools

import jax
import jax.numpy as jnp
from jax import lax
from jax.experimental import pallas as pl
from jax.experimental.pallas import tpu as pltpu
from jax.experimental.pallas import tpu_sc as plsc

N = 50000
D = 256
S = 512
NC, NS = 2, 16
NW = NC * NS
SEG_T = S // NW
P = 128
NV = D // 16
BS_STEPS = 16


def kernel(x, batch):
    batch_i32 = batch.astype(jnp.int32)
    mesh = plsc.VectorSubcoreMesh(core_axis_name="c", subcore_axis_name="s")

    @functools.partial(
        pl.kernel,
        mesh=mesh,
        compiler_params=pltpu.CompilerParams(needs_layout_passes=False),
        out_type=jax.ShapeDtypeStruct((S, D), jnp.float32),
        scratch_types=[
            pltpu.VMEM((N,), jnp.int32),
            pltpu.VMEM((P, D), jnp.float32),
            pltpu.VMEM((P, D), jnp.float32),
            pltpu.VMEM((SEG_T, D), jnp.float32),
            pltpu.SemaphoreType.DMA,
            pltpu.SemaphoreType.DMA,
        ],
    )
    def k(x_hbm, b_hbm, out_hbm, batch_v, buf_a, buf_b, acc_v, sem_a, sem_b):
        c = lax.axis_index("c")
        s = lax.axis_index("s")
        w = c * NS + s
        seg0 = w * SEG_T

        pltpu.sync_copy(b_hbm, batch_v)

        lane = lax.iota(jnp.int32, 16)

        def lower_bound(tgt):
            def step(_, lh):
                lo, hi = lh
                active = lo < hi
                mid = jnp.minimum((lo + hi) // 2, N - 1)
                vals = plsc.load_gather(batch_v, [mid])
                pred = vals < tgt
                lo = jnp.where(active & pred, mid + 1, lo)
                hi = jnp.where(active & (~pred), mid, hi)
                return lo, hi
            lo, _ = lax.fori_loop(
                0, BS_STEPS, step,
                (jnp.zeros((16,), jnp.int32), jnp.full((16,), N, jnp.int32)))
            return lo

        lob = lower_bound(seg0 + lane)
        upb = lower_bound(seg0 + 1 + lane)
        recips = jnp.ones((16,), jnp.float32) / jnp.maximum(
            (upb - lob).astype(jnp.float32), 1.0)

        def extract(vec, idx):
            return jnp.sum(jnp.where(lane == idx, vec, 0))

        los = [extract(lob, kk) for kk in range(SEG_T)]
        ns = [extract(upb - lob, kk) for kk in range(SEG_T)]

        def aligned(base):
            return pl.multiple_of(
                jnp.minimum((base // 8) * 8, N - P), 8)

        bufs = (buf_a, buf_b)
        sems = (sem_a, sem_b)

        zero = jnp.zeros((16,), jnp.float32)
        handle = pltpu.async_copy(
            x_hbm.at[pl.ds(aligned(los[0]), P)], buf_a, sem_a)
        for kseg in range(SEG_T):
            if kseg + 1 < SEG_T:
                next_handle = pltpu.async_copy(
                    x_hbm.at[pl.ds(aligned(los[kseg + 1]), P)],
                    bufs[(kseg + 1) % 2], sems[(kseg + 1) % 2])
            handle.wait()
            buf = bufs[kseg % 2]

            a0 = aligned(los[kseg])
            off0 = los[kseg] - a0
            m0 = jnp.minimum(ns[kseg], P - off0)

            def make_row(b, off):
                def row(i, a):
                    r = off + i
                    return tuple(a[j] + b[r, pl.ds(j * 16, 16)]
                                 for j in range(NV))
                return row

            acc = lax.fori_loop(0, m0, make_row(buf, off0), (zero,) * NV)

            def more_cond(st):
                return st[0] < ns[kseg]

            def more_body(st):
                done = st[0]
                a2 = aligned(los[kseg] + done)
                off2 = los[kseg] + done - a2
                pltpu.sync_copy(x_hbm.at[pl.ds(a2, P)], buf)
                m = jnp.minimum(ns[kseg] - done, P - off2)
                acc2 = lax.fori_loop(0, m, make_row(buf, off2), st[1:])
                return (done + m,) + acc2

            st = lax.while_loop(more_cond, more_body, (m0,) + acc)

            recip = jnp.full((16,), extract(recips, kseg))
            for j in range(NV):
                acc_v[kseg, pl.ds(j * 16, 16)] = st[1 + j] * recip

            if kseg + 1 < SEG_T:
                handle = next_handle

        pltpu.sync_copy(acc_v, out_hbm.at[pl.ds(pl.multiple_of(seg0, 8),
                                                SEG_T)])

    return k(x, batch_i32)


# speedup vs baseline: 9.5448x; 1.2472x over previous
import functools

import jax
import jax.numpy as jnp
from jax import lax
from jax.experimental import pallas as pl
from jax.experimental.pallas import tpu as pltpu
from jax.experimental.pallas import tpu_sc as plsc

N = 50000
D = 256
S = 512
NC, NS = 2, 16
NW = NC * NS
SEG_T = S // NW
C = 128
NV = D // 16
BS_STEPS = 16


def kernel(x, batch):
    batch_i32 = batch.astype(jnp.int32)
    mesh = plsc.VectorSubcoreMesh(core_axis_name="c", subcore_axis_name="s")

    @functools.partial(
        pl.kernel,
        mesh=mesh,
        compiler_params=pltpu.CompilerParams(needs_layout_passes=False),
        out_type=jax.ShapeDtypeStruct((S, D), jnp.float32),
        scratch_types=[
            pltpu.VMEM((N,), jnp.int32),
            pltpu.VMEM((C, D), jnp.float32),
            pltpu.VMEM((C, D), jnp.float32),
            pltpu.VMEM((SEG_T, D), jnp.float32),
            pltpu.SMEM((SEG_T + 1,), jnp.int32),
            pltpu.SemaphoreType.DMA,
            pltpu.SemaphoreType.DMA,
        ],
    )
    def k(x_hbm, b_hbm, out_hbm, batch_v, buf_a, buf_b, acc_v, bnd_s,
          sem_a, sem_b):
        c = lax.axis_index("c")
        s = lax.axis_index("s")
        w = c * NS + s
        seg0 = w * SEG_T

        pltpu.sync_copy(b_hbm, batch_v)

        lane = lax.iota(jnp.int32, 16)

        def lower_bound(tgt):
            def step(_, lh):
                lo, hi = lh
                active = lo < hi
                mid = jnp.minimum((lo + hi) // 2, N - 1)
                vals = plsc.load_gather(batch_v, [mid])
                pred = vals < tgt
                lo = jnp.where(active & pred, mid + 1, lo)
                hi = jnp.where(active & (~pred), mid, hi)
                return lo, hi
            lo, _ = lax.fori_loop(
                0, BS_STEPS, step,
                (jnp.zeros((16,), jnp.int32), jnp.full((16,), N, jnp.int32)))
            return lo

        lob = lower_bound(seg0 + lane)
        upb = lower_bound(seg0 + 1 + lane)
        recips = jnp.ones((16,), jnp.float32) / jnp.maximum(
            (upb - lob).astype(jnp.float32), 1.0)

        def extract(vec, idx):
            return jnp.sum(jnp.where(lane == idx, vec, 0))

        lo_w = extract(lob, 0)
        hi_w = extract(upb, SEG_T - 1)
        bnd_s[0] = lo_w
        for kk in range(SEG_T):
            bnd_s[kk + 1] = extract(upb, kk)

        a0 = pl.multiple_of(jnp.minimum((lo_w // 8) * 8, N - C), 8)
        nch = (hi_w - a0 + C - 1) // C

        def chunk_base(i):
            return pl.multiple_of(
                jnp.minimum(a0 + i * C, N - C), 8)

        bufs = (buf_a, buf_b)
        sems = (sem_a, sem_b)

        def issue(i, parity):
            @pl.when(i < nch)
            def _():
                pltpu.async_copy(x_hbm.at[pl.ds(chunk_base(i), C)],
                                 bufs[parity], sems[parity])

        def drain(parity):
            pltpu.make_async_copy(x_hbm.at[pl.ds(0, C)], bufs[parity],
                                  sems[parity]).wait()

        issue(0, 0)
        issue(1, 1)

        zero = jnp.zeros((16,), jnp.float32)

        def process(i, parity, st):
            buf = bufs[parity]
            bc = chunk_base(i)
            ce = jnp.minimum(hi_w, bc + C)

            def piece_cond(pst):
                return pst[0] < ce

            def piece_body(pst):
                p, kseg = pst[0], pst[1]
                acc = pst[2:]
                b_next = bnd_s[kseg + 1]
                e = jnp.minimum(b_next, ce)

                def row(r, a):
                    return tuple(a[j] + buf[r - bc, pl.ds(j * 16, 16)]
                                 for j in range(NV))
                acc = lax.fori_loop(p, e, row, acc)
                flushed = e == b_next

                @pl.when(flushed)
                def _():
                    recip = jnp.full((16,), extract(recips, kseg))
                    for j in range(NV):
                        acc_v[kseg, pl.ds(j * 16, 16)] = acc[j] * recip

                acc = tuple(jnp.where(flushed, zero, a) for a in acc)
                return (e, kseg + flushed.astype(jnp.int32)) + acc

            return lax.while_loop(piece_cond, piece_body, st)

        def loop_cond(st):
            return st[0] < nch

        def loop_body(st):
            i = st[0]
            drain(0)
            pst = process(i, 0, st[1:])
            issue(i + 2, 0)

            @pl.when(i + 1 < nch)
            def _():
                drain(1)
            pst = process(i + 1, 1, pst)
            issue(i + 3, 1)
            return (i + 2,) + pst

        st = lax.while_loop(loop_cond, loop_body,
                            (0, lo_w, 0) + (zero,) * NV)
        kseg_end = st[2]

        def tail_cond(kk):
            return kk < SEG_T

        def tail_body(kk):
            for j in range(NV):
                acc_v[kk, pl.ds(j * 16, 16)] = zero
            return kk + 1

        lax.while_loop(tail_cond, tail_body, kseg_end)

        pltpu.sync_copy(acc_v, out_hbm.at[pl.ds(pl.multiple_of(seg0, 8),
                                                SEG_T)])

    return k(x, batch_i32)
